# baseline (speedup 1.0000x reference)
import numpy as np
import jax
import jax.numpy as jnp
from jax.experimental import pallas as pl


def _vtab():
    m = np.zeros((21, 37), dtype=np.float32)
    m[:, :4] = 1.0
    for r in range(21):
        m[r, 4:4 + (r % 8) + 1] = 1.0
    return m


_VT = _vtab()
_S = int(np.nonzero(_VT.max(axis=0))[0].max()) + 1 - 3
_B, _L, _D = 2, 128, 128
_M = _B * _L * _S
_BJ = 256
_NT = _M // _BJ

_VA_T = np.tile(np.ascontiguousarray(_VT[:, 3:3 + _S].T), (_B * _L, 1))

_HIGH = jax.lax.Precision.HIGHEST


def _body(aa_ref, aa9_ref, se_ref, te_ref, wq_ref, pco_ref, pda_ref,
          pdb_ref, va_ref, m9_ref, vt_ref, mc_ref, out_ref, am_ref):
    f32 = jnp.float32
    n_bl = _B * _L

    def first_argmax(a, n):
        rows = a.shape[0]
        mx = jnp.max(a, axis=-1, keepdims=True)
        idx = jax.lax.broadcasted_iota(jnp.int32, (rows, n), 1)
        return jnp.min(jnp.where(a == mx, idx, jnp.int32(n)), axis=-1,
                       keepdims=True)

    rt = first_argmax(aa_ref[...], 20)
    oh = (rt == jax.lax.broadcasted_iota(jnp.int32, (n_bl, 21), 1)).astype(f32)
    am37 = jax.lax.dot_general(oh, vt_ref[...], (((1,), (0,)), ((), ())),
                               precision=_HIGH)
    am_ref[...] = am37 * mc_ref[...]

    rt9 = first_argmax(aa9_ref[...], 20)
    ohm = (rt9 == jax.lax.broadcasted_iota(jnp.int32, (_M, 21), 1)).astype(f32)
    vm = jnp.sum(ohm * va_ref[...], axis=-1, keepdims=True) * m9_ref[...]

    h = se_ref[...] + te_ref[...] * vm
    qs = jax.lax.dot_general(h, wq_ref[...], (((1,), (0,)), ((), ())),
                             precision=_HIGH) / jnp.sqrt(f32(_D))

    pco = pco_ref[...]
    pda = pda_ref[...]

    for t in range(_NT):
        j0 = t * _BJ
        h_j = h[j0:j0 + _BJ, :]
        logit = jax.lax.dot_general(qs, h_j, (((1,), (1,)), ((), ())),
                                    precision=_HIGH)
        dn = jax.lax.dot_general(pda, pdb_ref[:, j0:j0 + _BJ],
                                 (((1,), (0,)), ((), ())),
                                 precision=_HIGH)
        w = jnp.where(dn <= 0.0, jax.nn.sigmoid(logit), 0.0) * vm
        red = jax.lax.dot_general(w, pco, (((0,), (0,)), ((), ())),
                                  precision=_HIGH)
        pj = pco[j0:j0 + _BJ, :]
        vm_j = vm[j0:j0 + _BJ, :]
        colsum = red[:, 3:4]
        out_ref[j0:j0 + _BJ, :] = (
            pj + (red - pj * colsum) / (colsum + 1e-6)) * vm_j


def kernel(bb_pred, scalar_features, aa_pred, residue_batch, mask,
           atom_table, Wq):
    f32 = jnp.float32
    B, L, S, M = _B, _L, _S, _M
    maskf = mask.astype(f32)

    noise = jax.random.normal(jax.random.key(1), (B, L, 34, 3), f32) * 0.5
    ca = bb_pred[:, :, 1, :] * maskf[..., None]
    pos = (ca[:, :, None, :] + noise[:, :, :S, :]).reshape(M, 3)
    batch = jnp.where(mask, residue_batch.reshape(B, L), 0).astype(f32)
    batch9 = jnp.broadcast_to(batch[:, :, None], (B, L, S)).reshape(M, 1)

    ones = jnp.ones((M, 1), f32)
    zero3 = jnp.zeros((M, 3), f32)
    pco = jnp.concatenate([pos, ones, zero3, jnp.zeros((M, 1), f32)], axis=1)
    posb = pos.at[:, 0].add(batch9[:, 0] * 100.0)
    pn = jnp.sum(posb * posb, axis=1, keepdims=True)
    pda = jnp.concatenate([posb, ones, pn - 64.0, zero3], axis=1)
    pdb = jnp.concatenate([-2.0 * posb, pn, ones, zero3], axis=1).T

    aa2 = aa_pred.reshape(B * L, 20)
    aa9 = jnp.repeat(aa2, S, axis=0)
    se = jnp.repeat(scalar_features.reshape(B * L, _D), S, axis=0)
    te = jnp.tile(atom_table[3:3 + S, :], (B * L, 1))
    m9 = jnp.repeat(maskf.reshape(B * L), S)[:, None]
    mc = maskf.reshape(B * L, 1)

    out, am = pl.pallas_call(
        _body,
        out_shape=(
            jax.ShapeDtypeStruct((M, 8), f32),
            jax.ShapeDtypeStruct((B * L, 37), f32),
        ),
    )(aa2, aa9, se, te, Wq, pco, pda, pdb,
      jnp.asarray(_VA_T), m9, jnp.asarray(_VT), mc)

    sc = out[:, 0:3].reshape(B, L, S, 3)
    bb = bb_pred[:, :, 0:3, :] * maskf[:, :, None, None]
    coords = jnp.concatenate(
        [bb, sc, jnp.zeros((B, L, 37 - 3 - S, 3), f32)], axis=2)
    return coords, am.reshape(B, L, 37)


# speedup vs baseline: 1.3570x; 1.3570x over previous
import numpy as np
import jax
import jax.numpy as jnp
from jax.experimental import pallas as pl


def _vtab():
    m = np.zeros((21, 37), dtype=np.float32)
    m[:, :4] = 1.0
    for r in range(21):
        m[r, 4:4 + (r % 8) + 1] = 1.0
    return m


_VT = _vtab()
_S = int(np.nonzero(_VT.max(axis=0))[0].max()) + 1 - 3
_B, _L, _D = 2, 128, 128
_M = _B * _L * _S
_BJ = 256
_NT = _M // _BJ

_VA_T = np.tile(np.ascontiguousarray(_VT[:, 3:3 + _S].T), (_B * _L, 1))

_HIGH = jax.lax.Precision.HIGHEST


def _body(aa_ref, aa9_ref, se_ref, te_ref, wq_ref, pco_ref, pdc_ref,
          pdr_ref, va_ref, m9_ref, vt_ref, mc_ref, out_ref, am_ref):
    f32 = jnp.float32
    n_bl = _B * _L

    def first_argmax(a, n):
        rows = a.shape[0]
        mx = jnp.max(a, axis=-1, keepdims=True)
        idx = jax.lax.broadcasted_iota(jnp.int32, (rows, n), 1)
        return jnp.min(jnp.where(a == mx, idx, jnp.int32(n)), axis=-1,
                       keepdims=True)

    rt = first_argmax(aa_ref[...], 20)
    oh = (rt == jax.lax.broadcasted_iota(jnp.int32, (n_bl, 21), 1)).astype(f32)
    am37 = jax.lax.dot_general(oh, vt_ref[...], (((1,), (0,)), ((), ())),
                               precision=_HIGH)
    am_ref[...] = am37 * mc_ref[...]

    rt9 = first_argmax(aa9_ref[...], 20)
    ohm = (rt9 == jax.lax.broadcasted_iota(jnp.int32, (_M, 21), 1)).astype(f32)
    vm = jnp.sum(ohm * va_ref[...], axis=-1, keepdims=True) * m9_ref[...]

    h = se_ref[...] + te_ref[...] * vm
    qs = jax.lax.dot_general(h, wq_ref[...], (((1,), (0,)), ((), ())),
                             precision=_HIGH) / jnp.sqrt(f32(_D))

    pco = pco_ref[...]
    pdc = pdc_ref[...]
    px_c, py_c, pz_c = pdc[:, 0:1], pdc[:, 1:2], pdc[:, 2:3]

    for t in range(_NT):
        j0 = t * _BJ
        h_j = h[j0:j0 + _BJ, :]
        logit = jax.lax.dot_general(qs, h_j, (((1,), (1,)), ((), ())),
                                    precision=_HIGH)
        prs = pdr_ref[:, j0:j0 + _BJ]
        dx = px_c - prs[0:1, :]
        dy = py_c - prs[1:2, :]
        dz = pz_c - prs[2:3, :]
        dist2 = dx * dx + dy * dy + dz * dz
        w = jnp.where(dist2 <= 64.0, jax.nn.sigmoid(logit), 0.0) * vm
        red = jax.lax.dot_general(w, pco, (((0,), (0,)), ((), ())),
                                  precision=_HIGH)
        pj = pco[j0:j0 + _BJ, :]
        vm_j = vm[j0:j0 + _BJ, :]
        colsum = red[:, 3:4]
        out_ref[j0:j0 + _BJ, :] = (
            pj + (red - pj * colsum) / (colsum + 1e-6)) * vm_j


def kernel(bb_pred, scalar_features, aa_pred, residue_batch, mask,
           atom_table, Wq):
    f32 = jnp.float32
    B, L, S, M = _B, _L, _S, _M
    maskf = mask.astype(f32)

    noise = jax.random.normal(jax.random.key(1), (B, L, 34, 3), f32) * 0.5
    ca = bb_pred[:, :, 1, :] * maskf[..., None]
    pos = (ca[:, :, None, :] + noise[:, :, :S, :]).reshape(M, 3)
    batch = jnp.where(mask, residue_batch.reshape(B, L), 0).astype(f32)
    batch9 = jnp.broadcast_to(batch[:, :, None], (B, L, S)).reshape(M, 1)

    ones = jnp.ones((M, 1), f32)
    zero3 = jnp.zeros((M, 3), f32)
    pco = jnp.concatenate([pos, ones, zero3, jnp.zeros((M, 1), f32)], axis=1)
    posb = pos.at[:, 0].add(batch9[:, 0] * 100.0)
    pdc = jnp.concatenate([posb, ones, zero3, jnp.zeros((M, 1), f32)], axis=1)
    pdr = pdc.T

    aa2 = aa_pred.reshape(B * L, 20)
    aa9 = jnp.repeat(aa2, S, axis=0)
    se = jnp.repeat(scalar_features.reshape(B * L, _D), S, axis=0)
    te = jnp.tile(atom_table[3:3 + S, :], (B * L, 1))
    m9 = jnp.repeat(maskf.reshape(B * L), S)[:, None]
    mc = maskf.reshape(B * L, 1)

    out, am = pl.pallas_call(
        _body,
        out_shape=(
            jax.ShapeDtypeStruct((M, 8), f32),
            jax.ShapeDtypeStruct((B * L, 37), f32),
        ),
    )(aa2, aa9, se, te, Wq, pco, pdc, pdr,
      jnp.asarray(_VA_T), m9, jnp.asarray(_VT), mc)

    sc = out[:, 0:3].reshape(B, L, S, 3)
    bb = bb_pred[:, :, 0:3, :] * maskf[:, :, None, None]
    coords = jnp.concatenate(
        [bb, sc, jnp.zeros((B, L, 37 - 3 - S, 3), f32)], axis=2)
    return coords, am.reshape(B, L, 37)


# speedup vs baseline: 1.7004x; 1.2531x over previous
import numpy as np
import jax
import jax.numpy as jnp
from jax.experimental import pallas as pl


def _vtab():
    m = np.zeros((21, 37), dtype=np.float32)
    m[:, :4] = 1.0
    for r in range(21):
        m[r, 4:4 + (r % 8) + 1] = 1.0
    return m


_VT = _vtab()
_S = int(np.nonzero(_VT.max(axis=0))[0].max()) + 1 - 3
_B, _L, _D = 2, 128, 128
_M = _B * _L * _S
_BJ = 256
_NT = _M // _BJ

_VA_T = np.tile(np.ascontiguousarray(_VT[:, 3:3 + _S].T), (_B * _L, 1))

_HIGH = jax.lax.Precision.HIGHEST


def _body(aa_ref, aa9_ref, se_ref, te_ref, wq_ref, pco_ref, pdc_ref,
          pdr_ref, va_ref, m9_ref, vt_ref, mc_ref, out_ref, am_ref):
    f32 = jnp.float32
    n_bl = _B * _L

    def first_argmax(a, n):
        rows = a.shape[0]
        mx = jnp.max(a, axis=-1, keepdims=True)
        idx = jax.lax.broadcasted_iota(jnp.int32, (rows, n), 1)
        return jnp.min(jnp.where(a == mx, idx, jnp.int32(n)), axis=-1,
                       keepdims=True)

    rt = first_argmax(aa_ref[...], 20)
    oh = (rt == jax.lax.broadcasted_iota(jnp.int32, (n_bl, 21), 1)).astype(f32)
    am37 = jax.lax.dot_general(oh, vt_ref[...], (((1,), (0,)), ((), ())),
                               precision=_HIGH)
    am_ref[...] = am37 * mc_ref[...]

    rt9 = first_argmax(aa9_ref[...], 20)
    ohm = (rt9 == jax.lax.broadcasted_iota(jnp.int32, (_M, 21), 1)).astype(f32)
    vm = jnp.sum(ohm * va_ref[...], axis=-1, keepdims=True) * m9_ref[...]

    h = se_ref[...] + te_ref[...] * vm
    qs = jax.lax.dot_general(h, wq_ref[...], (((1,), (0,)), ((), ())),
                             precision=_HIGH) / jnp.sqrt(f32(_D))

    pco = pco_ref[...]
    pdc = pdc_ref[...]
    px_c, py_c, pz_c = pdc[:, 0:1], pdc[:, 1:2], pdc[:, 2:3]

    for t in range(_NT):
        j0 = t * _BJ
        h_j = h[j0:j0 + _BJ, :]
        logit = jax.lax.dot_general(qs, h_j, (((1,), (1,)), ((), ())),
                                    precision=jax.lax.Precision.DEFAULT)
        prs = pdr_ref[:, j0:j0 + _BJ]
        dx = px_c - prs[0:1, :]
        dy = py_c - prs[1:2, :]
        dz = pz_c - prs[2:3, :]
        dist2 = dx * dx + dy * dy + dz * dz
        w = jnp.where(dist2 <= 64.0, jax.nn.sigmoid(logit), 0.0) * vm
        red = jax.lax.dot_general(w, pco, (((0,), (0,)), ((), ())),
                                  precision=_HIGH)
        pj = pco[j0:j0 + _BJ, :]
        vm_j = vm[j0:j0 + _BJ, :]
        colsum = red[:, 3:4]
        out_ref[j0:j0 + _BJ, :] = (
            pj + (red - pj * colsum) / (colsum + 1e-6)) * vm_j


def kernel(bb_pred, scalar_features, aa_pred, residue_batch, mask,
           atom_table, Wq):
    f32 = jnp.float32
    B, L, S, M = _B, _L, _S, _M
    maskf = mask.astype(f32)

    noise = jax.random.normal(jax.random.key(1), (B, L, 34, 3), f32) * 0.5
    ca = bb_pred[:, :, 1, :] * maskf[..., None]
    pos = (ca[:, :, None, :] + noise[:, :, :S, :]).reshape(M, 3)
    batch = jnp.where(mask, residue_batch.reshape(B, L), 0).astype(f32)
    batch9 = jnp.broadcast_to(batch[:, :, None], (B, L, S)).reshape(M, 1)

    ones = jnp.ones((M, 1), f32)
    zero3 = jnp.zeros((M, 3), f32)
    pco = jnp.concatenate([pos, ones, zero3, jnp.zeros((M, 1), f32)], axis=1)
    posb = pos.at[:, 0].add(batch9[:, 0] * 100.0)
    pdc = jnp.concatenate([posb, ones, zero3, jnp.zeros((M, 1), f32)], axis=1)
    pdr = pdc.T

    aa2 = aa_pred.reshape(B * L, 20)
    aa9 = jnp.repeat(aa2, S, axis=0)
    se = jnp.repeat(scalar_features.reshape(B * L, _D), S, axis=0)
    te = jnp.tile(atom_table[3:3 + S, :], (B * L, 1))
    m9 = jnp.repeat(maskf.reshape(B * L), S)[:, None]
    mc = maskf.reshape(B * L, 1)

    out, am = pl.pallas_call(
        _body,
        out_shape=(
            jax.ShapeDtypeStruct((M, 8), f32),
            jax.ShapeDtypeStruct((B * L, 37), f32),
        ),
    )(aa2, aa9, se, te, Wq, pco, pdc, pdr,
      jnp.asarray(_VA_T), m9, jnp.asarray(_VT), mc)

    sc = out[:, 0:3].reshape(B, L, S, 3)
    bb = bb_pred[:, :, 0:3, :] * maskf[:, :, None, None]
    coords = jnp.concatenate(
        [bb, sc, jnp.zeros((B, L, 37 - 3 - S, 3), f32)], axis=2)
    return coords, am.reshape(B, L, 37)


# speedup vs baseline: 2.2138x; 1.3019x over previous
import numpy as np
import jax
import jax.numpy as jnp
from jax.experimental import pallas as pl


def _vtab():
    m = np.zeros((21, 37), dtype=np.float32)
    m[:, :4] = 1.0
    for r in range(21):
        m[r, 4:4 + (r % 8) + 1] = 1.0
    return m


_VT = _vtab()
_S = int(np.nonzero(_VT.max(axis=0))[0].max()) + 1 - 3
_B, _L, _D = 2, 128, 128
_M = _B * _L * _S
_BJ = 256
_NT = _M // _BJ

_VA_T = np.tile(np.ascontiguousarray(_VT[:, 3:3 + _S].T), (_B * _L, 1))

_HIGH = jax.lax.Precision.HIGHEST


def _body(aa_ref, aa9_ref, se_ref, te_ref, wq_ref, pco_ref, pdc_ref,
          pdr_ref, va_ref, m9_ref, vt_ref, mc_ref, out_ref, am_ref):
    f32 = jnp.float32
    n_bl = _B * _L

    def first_argmax(a, n):
        rows = a.shape[0]
        mx = jnp.max(a, axis=-1, keepdims=True)
        idx = jax.lax.broadcasted_iota(jnp.int32, (rows, n), 1)
        return jnp.min(jnp.where(a == mx, idx, jnp.int32(n)), axis=-1,
                       keepdims=True)

    rt = first_argmax(aa_ref[...], 20)
    oh = (rt == jax.lax.broadcasted_iota(jnp.int32, (n_bl, 21), 1)).astype(f32)
    am37 = jax.lax.dot_general(oh, vt_ref[...], (((1,), (0,)), ((), ())),
                               precision=_HIGH)
    am_ref[...] = am37 * mc_ref[...]

    rt9 = first_argmax(aa9_ref[...], 20)
    ohm = (rt9 == jax.lax.broadcasted_iota(jnp.int32, (_M, 21), 1)).astype(f32)
    vm = jnp.sum(ohm * va_ref[...], axis=-1, keepdims=True) * m9_ref[...]

    h = se_ref[...] + te_ref[...] * vm
    qs = jax.lax.dot_general(h, wq_ref[...], (((1,), (0,)), ((), ())),
                             precision=jax.lax.Precision.DEFAULT
                             ) / jnp.sqrt(f32(_D))

    pco = pco_ref[...]
    pdc = pdc_ref[...]
    px_c, py_c, pz_c = pdc[:, 0:1], pdc[:, 1:2], pdc[:, 2:3]

    for t in range(_NT):
        j0 = t * _BJ
        h_j = h[j0:j0 + _BJ, :]
        logit = jax.lax.dot_general(qs, h_j, (((1,), (1,)), ((), ())),
                                    precision=jax.lax.Precision.DEFAULT)
        prs = pdr_ref[:, j0:j0 + _BJ]
        dx = px_c - prs[0:1, :]
        dy = py_c - prs[1:2, :]
        dz = pz_c - prs[2:3, :]
        dist2 = dx * dx + dy * dy + dz * dz
        w = jnp.where(dist2 <= 64.0, jax.nn.sigmoid(logit), 0.0) * vm
        red = jax.lax.dot_general(w, pco, (((0,), (0,)), ((), ())),
                                  precision=jax.lax.Precision.DEFAULT)
        pj = pco[j0:j0 + _BJ, :]
        vm_j = vm[j0:j0 + _BJ, :]
        colsum = red[:, 3:4]
        out_ref[j0:j0 + _BJ, :] = (
            pj + (red - pj * colsum) / (colsum + 1e-6)) * vm_j


def kernel(bb_pred, scalar_features, aa_pred, residue_batch, mask,
           atom_table, Wq):
    f32 = jnp.float32
    B, L, S, M = _B, _L, _S, _M
    maskf = mask.astype(f32)

    noise = jax.random.normal(jax.random.key(1), (B, L, 34, 3), f32) * 0.5
    ca = bb_pred[:, :, 1, :] * maskf[..., None]
    pos = (ca[:, :, None, :] + noise[:, :, :S, :]).reshape(M, 3)
    batch = jnp.where(mask, residue_batch.reshape(B, L), 0).astype(f32)
    batch9 = jnp.broadcast_to(batch[:, :, None], (B, L, S)).reshape(M, 1)

    ones = jnp.ones((M, 1), f32)
    zero3 = jnp.zeros((M, 3), f32)
    pco = jnp.concatenate([pos, ones, zero3, jnp.zeros((M, 1), f32)], axis=1)
    posb = pos.at[:, 0].add(batch9[:, 0] * 100.0)
    pdc = jnp.concatenate([posb, ones, zero3, jnp.zeros((M, 1), f32)], axis=1)
    pdr = pdc.T

    aa2 = aa_pred.reshape(B * L, 20)
    aa9 = jnp.repeat(aa2, S, axis=0)
    se = jnp.repeat(scalar_features.reshape(B * L, _D), S, axis=0)
    te = jnp.tile(atom_table[3:3 + S, :], (B * L, 1))
    m9 = jnp.repeat(maskf.reshape(B * L), S)[:, None]
    mc = maskf.reshape(B * L, 1)

    out, am = pl.pallas_call(
        _body,
        out_shape=(
            jax.ShapeDtypeStruct((M, 8), f32),
            jax.ShapeDtypeStruct((B * L, 37), f32),
        ),
    )(aa2, aa9, se, te, Wq, pco, pdc, pdr,
      jnp.asarray(_VA_T), m9, jnp.asarray(_VT), mc)

    sc = out[:, 0:3].reshape(B, L, S, 3)
    bb = bb_pred[:, :, 0:3, :] * maskf[:, :, None, None]
    coords = jnp.concatenate(
        [bb, sc, jnp.zeros((B, L, 37 - 3 - S, 3), f32)], axis=2)
    return coords, am.reshape(B, L, 37)
